# baseline (speedup 1.0000x reference)
import dataclasses
import functools

import jax
import jax.numpy as jnp
import numpy as np
from jax import lax
from jax.experimental import pallas as pl
from jax.experimental.pallas import tpu as pltpu
from jax.experimental.pallas import tpu_sc as plsc

EMBED = 256
HEADS = 8
POINTS = 4
NCAM = 4
FH = 64
FW = 176
HW = FH * FW
NQ = 16384
NLANE = 512
NPAIR = NQ * HEADS
DH = EMBED // HEADS

BQ = 1024
NWORK = 32
QCH = 2
NCHUNK = NQ // QCH
CHPW = NCHUNK // NWORK


def _selection_matrices():
    L = np.arange(NLANE)
    corner, h, cam, p = L >> 7, (L >> 4) & 7, (L >> 2) & 3, L & 3
    sx = np.zeros((HEADS * POINTS * 2, NLANE), np.float32)
    sy = np.zeros((HEADS * POINTS * 2, NLANE), np.float32)
    sx[h * POINTS * 2 + p * 2 + 0, L] = 1.0
    sy[h * POINTS * 2 + p * 2 + 1, L] = 1.0
    saw = np.zeros((HEADS * POINTS, NLANE), np.float32)
    saw[h * POINTS + p, L] = 1.0
    rx = np.zeros((NCAM * 2, NLANE), np.float32)
    ry = np.zeros((NCAM * 2, NLANE), np.float32)
    rx[cam * 2 + 0, L] = 1.0
    ry[cam * 2 + 1, L] = 1.0
    pool = np.kron(np.eye(HEADS, dtype=np.float32), np.ones((POINTS, POINTS), np.float32))
    return sx, sy, saw, rx, ry, pool


def _prep_body(q_ref, qp_ref, ref_ref, wso_ref, bso_ref, waw_ref, baw_ref,
               sx_ref, sy_ref, saw_ref, rx_ref, ry_ref, pool_ref,
               idx_ref, wt_ref):
    hp = jax.lax.Precision.HIGHEST
    q = q_ref[...] + qp_ref[...]
    so = jnp.dot(q, wso_ref[...], preferred_element_type=jnp.float32, precision=hp) + bso_ref[...]
    awl = jnp.dot(q, waw_ref[...], preferred_element_type=jnp.float32, precision=hp) + baw_ref[...]
    awl = awl - jnp.max(awl, axis=-1, keepdims=True)
    aw_e = jnp.exp(awl)
    aw = aw_e / jnp.dot(aw_e, pool_ref[...], preferred_element_type=jnp.float32, precision=hp)

    sx512 = jnp.dot(so, sx_ref[...], preferred_element_type=jnp.float32, precision=hp)
    sy512 = jnp.dot(so, sy_ref[...], preferred_element_type=jnp.float32, precision=hp)
    aw512 = jnp.dot(aw, saw_ref[...], preferred_element_type=jnp.float32, precision=hp)
    refs = ref_ref[...]
    rx512 = jnp.dot(refs, rx_ref[...], preferred_element_type=jnp.float32, precision=hp)
    ry512 = jnp.dot(refs, ry_ref[...], preferred_element_type=jnp.float32, precision=hp)

    lane = lax.broadcasted_iota(jnp.int32, (BQ, NLANE), 1)
    cxf = ((lane >> 7) & 1).astype(jnp.float32)
    cyf = ((lane >> 8) & 1).astype(jnp.float32)
    h = (lane >> 4) & 7
    cam = (lane >> 2) & 3

    x = rx512 * float(FW) + sx512 - 0.5
    y = ry512 * float(FH) + sy512 - 0.5
    xf = jnp.floor(x)
    yf = jnp.floor(y)
    fx = x - xf
    fy = y - yf
    xi = xf + cxf
    yi = yf + cyf
    wx = cxf * fx + (1.0 - cxf) * (1.0 - fx)
    wy = cyf * fy + (1.0 - cyf) * (1.0 - fy)
    valid = ((xi >= 0.0) & (xi <= float(FW - 1)) & (yi >= 0.0) & (yi <= float(FH - 1)))
    xi_c = jnp.clip(xi, 0.0, float(FW - 1)).astype(jnp.int32)
    yi_c = jnp.clip(yi, 0.0, float(FH - 1)).astype(jnp.int32)
    wt_ref[...] = aw512 * wx * wy * valid.astype(jnp.float32)
    idx_ref[...] = cam * (HW * HEADS) + (yi_c * FW + xi_c) * HEADS + h


def _vproj_body(v_ref, w_ref, b_ref, o_ref):
    hp = jax.lax.Precision.HIGHEST
    o_ref[...] = (jnp.dot(v_ref[...], w_ref[...], preferred_element_type=jnp.float32, precision=hp)
                  + b_ref[...]).astype(jnp.bfloat16)


def _out_body(m_ref, w_ref, b_ref, id_ref, o_ref):
    hp = jax.lax.Precision.HIGHEST
    o_ref[...] = (jnp.dot(m_ref[...], w_ref[...], preferred_element_type=jnp.float32, precision=hp)
                  + b_ref[...] + id_ref[...])


def _sc_kernel(idx_hbm, wt_hbm, table_hbm, out_hbm,
               idx_v0, idx_v1, wt_v0, wt_v1, g_v0, g_v1, out_v0, out_v1,
               sem_in0, sem_in1, sem_g0, sem_g1, sem_out0, sem_out1):
    cid = lax.axis_index("c")
    sid = lax.axis_index("s")
    wid = cid * 16 + sid
    base_ch = wid * CHPW

    idx_v = [idx_v0, idx_v1]
    wt_v = [wt_v0, wt_v1]
    g_v = [g_v0, g_v1]
    out_v = [out_v0, out_v1]
    sem_in = [sem_in0, sem_in1]
    sem_g = [sem_g0, sem_g1]
    sem_out = [sem_out0, sem_out1]

    def stage(ch, b):
        ch = jnp.minimum(ch, NCHUNK - 1)
        pltpu.async_copy(idx_hbm.at[pl.ds(ch * 2, 2)], idx_v[b], sem_in[b])
        pltpu.async_copy(wt_hbm.at[pl.ds(ch * 2, 2)], wt_v[b], sem_in[b])

    def wait_stage(b):
        pltpu.make_async_copy(idx_hbm.at[pl.ds(0, 2)], idx_v[b], sem_in[b]).wait()
        pltpu.make_async_copy(wt_hbm.at[pl.ds(0, 2)], wt_v[b], sem_in[b]).wait()

    def gath(b):
        for j in range(8):
            pltpu.async_copy(
                table_hbm.at[idx_v[b].at[j >> 2, pl.ds((j & 3) * 128, 128)]],
                g_v[b].at[pl.ds(j * 128, 128)], sem_g[b])

    def wait_gath(b):
        for j in range(8):
            pltpu.make_async_copy(table_hbm.at[pl.ds(0, 128)],
                                  g_v[b].at[pl.ds(j * 128, 128)], sem_g[b]).wait()

    def compute(b):
        wtb = wt_v[b]
        gvb = g_v[b]
        ovb = out_v[b]

        @pl.loop(0, 16)
        def _(i):
            hoff = (i & 7) * 16
            qq = i >> 3
            acc_e = jnp.zeros((16,), jnp.float32)
            acc_o = jnp.zeros((16,), jnp.float32)
            for corner in range(4):
                w16 = wtb[qq, pl.ds(corner * 128 + hoff, 16)]
                start = qq * NLANE + corner * 128 + hoff
                for k in range(16):
                    wk = lax.gather(
                        w16, jnp.full((16, 1), k, jnp.int32),
                        lax.GatherDimensionNumbers(
                            offset_dims=(), collapsed_slice_dims=(0,),
                            start_index_map=(0,)),
                        (1,), mode=lax.GatherScatterMode.PROMISE_IN_BOUNDS)
                    row = gvb[start + k, pl.ds(0, DH)]
                    ge, go = plsc.unpack(row, format=plsc.PackFormat.INTERLEAVED)
                    acc_e = acc_e + wk * ge
                    acc_o = acc_o + wk * go
            iot = lax.iota(jnp.int32, 16)
            rowi = jnp.full((16,), i, jnp.int32)
            plsc.store_scatter(ovb, [rowi, iot * 2], acc_e)
            plsc.store_scatter(ovb, [rowi, iot * 2 + 1], acc_o)

    stage(base_ch, 0)
    stage(base_ch + 1, 1)
    pltpu.async_copy(out_v0, out_hbm.at[pl.ds(base_ch * 16, 16)], sem_out0)
    pltpu.async_copy(out_v1, out_hbm.at[pl.ds(base_ch * 16 + 16, 16)], sem_out1)
    wait_stage(0)
    gath(0)

    @pl.loop(0, CHPW, step=2)
    def _(t):
        for b in range(2):
            ch = base_ch + t + b
            nb = 1 - b
            wait_stage(nb)
            gath(nb)
            wait_gath(b)
            pltpu.make_async_copy(out_v[b], out_hbm.at[pl.ds(0, 16)],
                                  sem_out[b]).wait()
            compute(b)
            pltpu.async_copy(out_v[b], out_hbm.at[pl.ds(ch * 16, 16)], sem_out[b])
            stage(ch + 2, b)

    wait_stage(1)
    wait_gath(0)
    pltpu.make_async_copy(out_v0, out_hbm.at[pl.ds(0, 16)], sem_out0).wait()
    pltpu.make_async_copy(out_v1, out_hbm.at[pl.ds(0, 16)], sem_out1).wait()


@jax.jit
def kernel(query, query_pos, value, reference_points, spatial_shapes,
           W_so, b_so, W_aw, b_aw, W_v, b_v, W_o, b_o):
    del spatial_shapes
    sx, sy, saw, rx, ry, pool = _selection_matrices()

    q2 = query.reshape(NQ, EMBED)
    qp2 = query_pos.reshape(NQ, EMBED)
    refs = reference_points.reshape(NQ, NCAM * 2)

    grid_a = NQ // BQ
    full = lambda shape: pl.BlockSpec(shape, lambda i: (0, 0))
    idx_all, wt_all = pl.pallas_call(
        _prep_body,
        grid=(grid_a,),
        in_specs=[
            pl.BlockSpec((BQ, EMBED), lambda i: (i, 0)),
            pl.BlockSpec((BQ, EMBED), lambda i: (i, 0)),
            pl.BlockSpec((BQ, NCAM * 2), lambda i: (i, 0)),
            full((EMBED, 64)), full((1, 64)),
            full((EMBED, 32)), full((1, 32)),
            full((64, NLANE)), full((64, NLANE)), full((32, NLANE)),
            full((8, NLANE)), full((8, NLANE)), full((32, 32)),
        ],
        out_specs=[
            pl.BlockSpec((BQ, NLANE), lambda i: (i, 0)),
            pl.BlockSpec((BQ, NLANE), lambda i: (i, 0)),
        ],
        out_shape=[
            jax.ShapeDtypeStruct((NQ, NLANE), jnp.int32),
            jax.ShapeDtypeStruct((NQ, NLANE), jnp.float32),
        ],
    )(q2, qp2, refs, W_so, b_so.reshape(1, 64), W_aw, b_aw.reshape(1, 32),
      jnp.asarray(sx), jnp.asarray(sy), jnp.asarray(saw),
      jnp.asarray(rx), jnp.asarray(ry), jnp.asarray(pool))

    v2 = value.reshape(NCAM * HW, EMBED)
    BV = 1024
    vproj = pl.pallas_call(
        _vproj_body,
        grid=(NCAM * HW // BV,),
        in_specs=[
            pl.BlockSpec((BV, EMBED), lambda i: (i, 0)),
            full((EMBED, EMBED)), full((1, EMBED)),
        ],
        out_specs=pl.BlockSpec((BV, EMBED), lambda i: (i, 0)),
        out_shape=jax.ShapeDtypeStruct((NCAM * HW, EMBED), jnp.bfloat16),
    )(v2, W_v, b_v.reshape(1, EMBED))

    table = vproj.reshape(NCAM * HW * HEADS, DH)

    mesh = plsc.VectorSubcoreMesh(core_axis_name="c", subcore_axis_name="s")
    cp = pltpu.CompilerParams(needs_layout_passes=False,
                              use_tc_tiling_on_sc=False)
    msda = pl.kernel(
        _sc_kernel,
        out_type=jax.ShapeDtypeStruct((NPAIR, DH), jnp.float32),
        mesh=mesh,
        scratch_types=[
            pltpu.VMEM((QCH, NLANE), jnp.int32),
            pltpu.VMEM((QCH, NLANE), jnp.int32),
            pltpu.VMEM((QCH, NLANE), jnp.float32),
            pltpu.VMEM((QCH, NLANE), jnp.float32),
            pltpu.VMEM((QCH * NLANE, DH), jnp.bfloat16),
            pltpu.VMEM((QCH * NLANE, DH), jnp.bfloat16),
            pltpu.VMEM((16, DH), jnp.float32),
            pltpu.VMEM((16, DH), jnp.float32),
            pltpu.SemaphoreType.DMA,
            pltpu.SemaphoreType.DMA,
            pltpu.SemaphoreType.DMA,
            pltpu.SemaphoreType.DMA,
            pltpu.SemaphoreType.DMA,
            pltpu.SemaphoreType.DMA,
        ],
        compiler_params=cp,
    )(idx_all, wt_all, table)

    m2 = msda.reshape(NQ, EMBED)
    out = pl.pallas_call(
        _out_body,
        grid=(NQ // BQ,),
        in_specs=[
            pl.BlockSpec((BQ, EMBED), lambda i: (i, 0)),
            full((EMBED, EMBED)), full((1, EMBED)),
            pl.BlockSpec((BQ, EMBED), lambda i: (i, 0)),
        ],
        out_specs=pl.BlockSpec((BQ, EMBED), lambda i: (i, 0)),
        out_shape=jax.ShapeDtypeStruct((NQ, EMBED), jnp.float32),
    )(m2, W_o, b_o.reshape(1, EMBED), q2)

    return out.reshape(1, NQ, EMBED)


# speedup vs baseline: 1.1132x; 1.1132x over previous
import dataclasses
import functools

import jax
import jax.numpy as jnp
import numpy as np
from jax import lax
from jax.experimental import pallas as pl
from jax.experimental.pallas import tpu as pltpu
from jax.experimental.pallas import tpu_sc as plsc

EMBED = 256
HEADS = 8
POINTS = 4
NCAM = 4
FH = 64
FW = 176
HW = FH * FW
NQ = 16384
NLANE = 512
NPAIR = NQ * HEADS
DH = EMBED // HEADS

BQ = 1024
NWORK = 32
QCH = 2
NCHUNK = NQ // QCH
CHPW = NCHUNK // NWORK


def _selection_matrices():
    L = np.arange(NLANE)
    corner, h, cam, p = L >> 7, (L >> 4) & 7, (L >> 2) & 3, L & 3
    sx = np.zeros((HEADS * POINTS * 2, NLANE), np.float32)
    sy = np.zeros((HEADS * POINTS * 2, NLANE), np.float32)
    sx[h * POINTS * 2 + p * 2 + 0, L] = 1.0
    sy[h * POINTS * 2 + p * 2 + 1, L] = 1.0
    saw = np.zeros((HEADS * POINTS, NLANE), np.float32)
    saw[h * POINTS + p, L] = 1.0
    rx = np.zeros((NCAM * 2, NLANE), np.float32)
    ry = np.zeros((NCAM * 2, NLANE), np.float32)
    rx[cam * 2 + 0, L] = 1.0
    ry[cam * 2 + 1, L] = 1.0
    pool = np.kron(np.eye(HEADS, dtype=np.float32), np.ones((POINTS, POINTS), np.float32))
    return sx, sy, saw, rx, ry, pool


def _prep_body(q_ref, qp_ref, ref_ref, wso_ref, bso_ref, waw_ref, baw_ref,
               sx_ref, sy_ref, saw_ref, rx_ref, ry_ref, pool_ref,
               idx_ref, wt_ref):
    hp = jax.lax.Precision.HIGHEST
    q = q_ref[...] + qp_ref[...]
    so = jnp.dot(q, wso_ref[...], preferred_element_type=jnp.float32, precision=hp) + bso_ref[...]
    awl = jnp.dot(q, waw_ref[...], preferred_element_type=jnp.float32) + baw_ref[...]
    awl = awl - jnp.max(awl, axis=-1, keepdims=True)
    aw_e = jnp.exp(awl)
    aw = aw_e / jnp.dot(aw_e, pool_ref[...], preferred_element_type=jnp.float32)

    sx512 = jnp.dot(so, sx_ref[...], preferred_element_type=jnp.float32, precision=hp)
    sy512 = jnp.dot(so, sy_ref[...], preferred_element_type=jnp.float32, precision=hp)
    aw512 = jnp.dot(aw, saw_ref[...], preferred_element_type=jnp.float32)
    refs = ref_ref[...]
    rx512 = jnp.dot(refs, rx_ref[...], preferred_element_type=jnp.float32, precision=hp)
    ry512 = jnp.dot(refs, ry_ref[...], preferred_element_type=jnp.float32, precision=hp)

    lane = lax.broadcasted_iota(jnp.int32, (BQ, NLANE), 1)
    cxf = ((lane >> 7) & 1).astype(jnp.float32)
    cyf = ((lane >> 8) & 1).astype(jnp.float32)
    h = (lane >> 4) & 7
    cam = (lane >> 2) & 3

    x = rx512 * float(FW) + sx512 - 0.5
    y = ry512 * float(FH) + sy512 - 0.5
    xf = jnp.floor(x)
    yf = jnp.floor(y)
    fx = x - xf
    fy = y - yf
    xi = xf + cxf
    yi = yf + cyf
    wx = cxf * fx + (1.0 - cxf) * (1.0 - fx)
    wy = cyf * fy + (1.0 - cyf) * (1.0 - fy)
    valid = ((xi >= 0.0) & (xi <= float(FW - 1)) & (yi >= 0.0) & (yi <= float(FH - 1)))
    xi_c = jnp.clip(xi, 0.0, float(FW - 1)).astype(jnp.int32)
    yi_c = jnp.clip(yi, 0.0, float(FH - 1)).astype(jnp.int32)
    wt_ref[...] = aw512 * wx * wy * valid.astype(jnp.float32)
    idx_ref[...] = cam * (HW * HEADS) + (yi_c * FW + xi_c) * HEADS + h


def _vproj_body(v_ref, w_ref, b_ref, o_ref):
    o_ref[...] = (jnp.dot(v_ref[...], w_ref[...], preferred_element_type=jnp.float32)
                  + b_ref[...])


def _out_body(m_ref, w_ref, b_ref, id_ref, o_ref):
    o_ref[...] = (jnp.dot(m_ref[...], w_ref[...], preferred_element_type=jnp.float32)
                  + b_ref[...] + id_ref[...])


def _sc_kernel(idx_hbm, wt_hbm, table_hbm, out_hbm,
               idx_v0, idx_v1, wt_v0, wt_v1, g_v0, g_v1, out_v0, out_v1,
               sem_in0, sem_in1, sem_g0, sem_g1, sem_out0, sem_out1):
    cid = lax.axis_index("c")
    sid = lax.axis_index("s")
    wid = cid * 16 + sid
    base_ch = wid * CHPW

    idx_v = [idx_v0, idx_v1]
    wt_v = [wt_v0, wt_v1]
    g_v = [g_v0, g_v1]
    out_v = [out_v0, out_v1]
    sem_in = [sem_in0, sem_in1]
    sem_g = [sem_g0, sem_g1]
    sem_out = [sem_out0, sem_out1]

    def stage(ch, b):
        ch = jnp.minimum(ch, NCHUNK - 1)
        pltpu.async_copy(idx_hbm.at[pl.ds(ch * 2, 2)], idx_v[b], sem_in[b])
        pltpu.async_copy(wt_hbm.at[pl.ds(ch * 2, 2)], wt_v[b], sem_in[b])

    def wait_stage(b):
        pltpu.make_async_copy(idx_hbm.at[pl.ds(0, 2)], idx_v[b], sem_in[b]).wait()
        pltpu.make_async_copy(wt_hbm.at[pl.ds(0, 2)], wt_v[b], sem_in[b]).wait()

    def gath(b):
        for j in range(8):
            pltpu.async_copy(
                table_hbm.at[idx_v[b].at[j >> 2, pl.ds((j & 3) * 128, 128)]],
                g_v[b].at[pl.ds(j * 128, 128)], sem_g[b])

    def wait_gath(b):
        for j in range(8):
            pltpu.make_async_copy(table_hbm.at[pl.ds(0, 128)],
                                  g_v[b].at[pl.ds(j * 128, 128)], sem_g[b]).wait()

    def compute(b):
        wtb = wt_v[b]
        gvb = g_v[b]
        ovb = out_v[b]

        @pl.loop(0, 16)
        def _(i):
            hoff = (i & 7) * 16
            qq = i >> 3
            acc0 = jnp.zeros((16,), jnp.float32)
            acc1 = jnp.zeros((16,), jnp.float32)
            for corner in range(4):
                w16 = wtb[qq, pl.ds(corner * 128 + hoff, 16)]
                start = qq * NLANE + corner * 128 + hoff
                for k in range(16):
                    wk = lax.gather(
                        w16, jnp.full((16, 1), k, jnp.int32),
                        lax.GatherDimensionNumbers(
                            offset_dims=(), collapsed_slice_dims=(0,),
                            start_index_map=(0,)),
                        (1,), mode=lax.GatherScatterMode.PROMISE_IN_BOUNDS)
                    g0 = gvb[start + k, pl.ds(0, 16)]
                    g1 = gvb[start + k, pl.ds(16, 16)]
                    acc0 = acc0 + wk * g0
                    acc1 = acc1 + wk * g1
            ovb[i, pl.ds(0, 16)] = acc0
            ovb[i, pl.ds(16, 16)] = acc1

    stage(base_ch, 0)
    stage(base_ch + 1, 1)
    pltpu.async_copy(out_v0, out_hbm.at[pl.ds(base_ch * 16, 16)], sem_out0)
    pltpu.async_copy(out_v1, out_hbm.at[pl.ds(base_ch * 16 + 16, 16)], sem_out1)
    wait_stage(0)
    gath(0)

    @pl.loop(0, CHPW, step=2)
    def _(t):
        for b in range(2):
            ch = base_ch + t + b
            nb = 1 - b
            wait_stage(nb)
            gath(nb)
            wait_gath(b)
            pltpu.make_async_copy(out_v[b], out_hbm.at[pl.ds(0, 16)],
                                  sem_out[b]).wait()
            compute(b)
            pltpu.async_copy(out_v[b], out_hbm.at[pl.ds(ch * 16, 16)], sem_out[b])
            stage(ch + 2, b)

    wait_stage(1)
    wait_gath(0)
    pltpu.make_async_copy(out_v0, out_hbm.at[pl.ds(0, 16)], sem_out0).wait()
    pltpu.make_async_copy(out_v1, out_hbm.at[pl.ds(0, 16)], sem_out1).wait()


@jax.jit
def kernel(query, query_pos, value, reference_points, spatial_shapes,
           W_so, b_so, W_aw, b_aw, W_v, b_v, W_o, b_o):
    del spatial_shapes
    sx, sy, saw, rx, ry, pool = _selection_matrices()

    q2 = query.reshape(NQ, EMBED)
    qp2 = query_pos.reshape(NQ, EMBED)
    refs = reference_points.reshape(NQ, NCAM * 2)

    grid_a = NQ // BQ
    full = lambda shape: pl.BlockSpec(shape, lambda i: (0, 0))
    idx_all, wt_all = pl.pallas_call(
        _prep_body,
        grid=(grid_a,),
        in_specs=[
            pl.BlockSpec((BQ, EMBED), lambda i: (i, 0)),
            pl.BlockSpec((BQ, EMBED), lambda i: (i, 0)),
            pl.BlockSpec((BQ, NCAM * 2), lambda i: (i, 0)),
            full((EMBED, 64)), full((1, 64)),
            full((EMBED, 32)), full((1, 32)),
            full((64, NLANE)), full((64, NLANE)), full((32, NLANE)),
            full((8, NLANE)), full((8, NLANE)), full((32, 32)),
        ],
        out_specs=[
            pl.BlockSpec((BQ, NLANE), lambda i: (i, 0)),
            pl.BlockSpec((BQ, NLANE), lambda i: (i, 0)),
        ],
        out_shape=[
            jax.ShapeDtypeStruct((NQ, NLANE), jnp.int32),
            jax.ShapeDtypeStruct((NQ, NLANE), jnp.float32),
        ],
    )(q2, qp2, refs, W_so, b_so.reshape(1, 64), W_aw, b_aw.reshape(1, 32),
      jnp.asarray(sx), jnp.asarray(sy), jnp.asarray(saw),
      jnp.asarray(rx), jnp.asarray(ry), jnp.asarray(pool))

    v2 = value.reshape(NCAM * HW, EMBED)
    BV = 1024
    vproj = pl.pallas_call(
        _vproj_body,
        grid=(NCAM * HW // BV,),
        in_specs=[
            pl.BlockSpec((BV, EMBED), lambda i: (i, 0)),
            full((EMBED, EMBED)), full((1, EMBED)),
        ],
        out_specs=pl.BlockSpec((BV, EMBED), lambda i: (i, 0)),
        out_shape=jax.ShapeDtypeStruct((NCAM * HW, EMBED), jnp.float32),
    )(v2, W_v, b_v.reshape(1, EMBED))

    table = vproj.reshape(NCAM * HW * HEADS, DH)

    mesh = plsc.VectorSubcoreMesh(core_axis_name="c", subcore_axis_name="s")
    cp = pltpu.CompilerParams(needs_layout_passes=False,
                              use_tc_tiling_on_sc=False)
    msda = pl.kernel(
        _sc_kernel,
        out_type=jax.ShapeDtypeStruct((NPAIR, DH), jnp.float32),
        mesh=mesh,
        scratch_types=[
            pltpu.VMEM((QCH, NLANE), jnp.int32),
            pltpu.VMEM((QCH, NLANE), jnp.int32),
            pltpu.VMEM((QCH, NLANE), jnp.float32),
            pltpu.VMEM((QCH, NLANE), jnp.float32),
            pltpu.VMEM((QCH * NLANE, DH), jnp.float32),
            pltpu.VMEM((QCH * NLANE, DH), jnp.float32),
            pltpu.VMEM((16, DH), jnp.float32),
            pltpu.VMEM((16, DH), jnp.float32),
            pltpu.SemaphoreType.DMA,
            pltpu.SemaphoreType.DMA,
            pltpu.SemaphoreType.DMA,
            pltpu.SemaphoreType.DMA,
            pltpu.SemaphoreType.DMA,
            pltpu.SemaphoreType.DMA,
        ],
        compiler_params=cp,
    )(idx_all, wt_all, table)

    m2 = msda.reshape(NQ, EMBED)
    out = pl.pallas_call(
        _out_body,
        grid=(NQ // BQ,),
        in_specs=[
            pl.BlockSpec((BQ, EMBED), lambda i: (i, 0)),
            full((EMBED, EMBED)), full((1, EMBED)),
            pl.BlockSpec((BQ, EMBED), lambda i: (i, 0)),
        ],
        out_specs=pl.BlockSpec((BQ, EMBED), lambda i: (i, 0)),
        out_shape=jax.ShapeDtypeStruct((NQ, EMBED), jnp.float32),
    )(m2, W_o, b_o.reshape(1, EMBED), q2)

    return out.reshape(1, NQ, EMBED)


# speedup vs baseline: 1.2035x; 1.0812x over previous
import dataclasses
import functools

import jax
import jax.numpy as jnp
import numpy as np
from jax import lax
from jax.experimental import pallas as pl
from jax.experimental.pallas import tpu as pltpu
from jax.experimental.pallas import tpu_sc as plsc

EMBED = 256
HEADS = 8
POINTS = 4
NCAM = 4
FH = 64
FW = 176
HW = FH * FW
NQ = 16384
NLANE = 512
NPAIR = NQ * HEADS
DH = EMBED // HEADS

BQ = 1024
NWORK = 32
QCH = 2
NCHUNK = NQ // QCH
CHPW = NCHUNK // NWORK


def _selection_matrices():
    L = np.arange(NLANE)
    corner, h, cam, p = L >> 7, (L >> 4) & 7, (L >> 2) & 3, L & 3
    sx = np.zeros((HEADS * POINTS * 2, NLANE), np.float32)
    sy = np.zeros((HEADS * POINTS * 2, NLANE), np.float32)
    sx[h * POINTS * 2 + p * 2 + 0, L] = 1.0
    sy[h * POINTS * 2 + p * 2 + 1, L] = 1.0
    saw = np.zeros((HEADS * POINTS, NLANE), np.float32)
    saw[h * POINTS + p, L] = 1.0
    rx = np.zeros((NCAM * 2, NLANE), np.float32)
    ry = np.zeros((NCAM * 2, NLANE), np.float32)
    rx[cam * 2 + 0, L] = 1.0
    ry[cam * 2 + 1, L] = 1.0
    pool = np.kron(np.eye(HEADS, dtype=np.float32), np.ones((POINTS, POINTS), np.float32))
    return sx, sy, saw, rx, ry, pool


def _prep_body(q_ref, qp_ref, ref_ref, wso_ref, bso_ref, waw_ref, baw_ref,
               sx_ref, sy_ref, saw_ref, rx_ref, ry_ref, pool_ref,
               i0, i1, i2, i3, w0, w1, w2, w3):
    idx_refs = (i0, i1, i2, i3)
    wt_refs = (w0, w1, w2, w3)
    hp = jax.lax.Precision.HIGHEST
    q = q_ref[...] + qp_ref[...]
    so = jnp.dot(q, wso_ref[...], preferred_element_type=jnp.float32, precision=hp) + bso_ref[...]
    awl = jnp.dot(q, waw_ref[...], preferred_element_type=jnp.float32) + baw_ref[...]
    awl = awl - jnp.max(awl, axis=-1, keepdims=True)
    aw_e = jnp.exp(awl)
    aw = aw_e / jnp.dot(aw_e, pool_ref[...], preferred_element_type=jnp.float32)

    sx512 = jnp.dot(so, sx_ref[...], preferred_element_type=jnp.float32, precision=hp)
    sy512 = jnp.dot(so, sy_ref[...], preferred_element_type=jnp.float32, precision=hp)
    aw512 = jnp.dot(aw, saw_ref[...], preferred_element_type=jnp.float32)
    refs = ref_ref[...]
    rx512 = jnp.dot(refs, rx_ref[...], preferred_element_type=jnp.float32, precision=hp)
    ry512 = jnp.dot(refs, ry_ref[...], preferred_element_type=jnp.float32, precision=hp)

    lane = lax.broadcasted_iota(jnp.int32, (BQ, NLANE), 1)
    cxf = ((lane >> 7) & 1).astype(jnp.float32)
    cyf = ((lane >> 8) & 1).astype(jnp.float32)
    h = (lane >> 4) & 7
    cam = (lane >> 2) & 3

    x = rx512 * float(FW) + sx512 - 0.5
    y = ry512 * float(FH) + sy512 - 0.5
    xf = jnp.floor(x)
    yf = jnp.floor(y)
    fx = x - xf
    fy = y - yf
    xi = xf + cxf
    yi = yf + cyf
    wx = cxf * fx + (1.0 - cxf) * (1.0 - fx)
    wy = cyf * fy + (1.0 - cyf) * (1.0 - fy)
    valid = ((xi >= 0.0) & (xi <= float(FW - 1)) & (yi >= 0.0) & (yi <= float(FH - 1)))
    xi_c = jnp.clip(xi, 0.0, float(FW - 1)).astype(jnp.int32)
    yi_c = jnp.clip(yi, 0.0, float(FH - 1)).astype(jnp.int32)
    wt512 = aw512 * wx * wy * valid.astype(jnp.float32)
    idx512 = cam * (HW * HEADS) + (yi_c * FW + xi_c) * HEADS + h
    for corner in range(4):
        sl = slice(corner * 128, (corner + 1) * 128)
        idx_refs[corner][...] = idx512[:, sl]
        wt_refs[corner][...] = wt512[:, sl]


def _vproj_body(v_ref, w_ref, b_ref, o_ref):
    o_ref[...] = (jnp.dot(v_ref[...], w_ref[...], preferred_element_type=jnp.float32)
                  + b_ref[...]).reshape(o_ref.shape)


def _out_body(m_ref, w_ref, b_ref, id_ref, o_ref):
    o_ref[...] = (jnp.dot(m_ref[...], w_ref[...], preferred_element_type=jnp.float32)
                  + b_ref[...] + id_ref[...])


def _sc_kernel(ix0, ix1, ix2, ix3, wx0, wx1, wx2, wx3, table_hbm, out_hbm,
               idx_v0, idx_v1, wt_v0, wt_v1, g_v0, g_v1, out_v0, out_v1,
               sem_in0, sem_in1, sem_g0, sem_g1, sem_out0, sem_out1):
    cid = lax.axis_index("c")
    sid = lax.axis_index("s")
    wid = cid * 16 + sid
    base_ch = wid * CHPW

    idx_v = [idx_v0, idx_v1]
    wt_v = [wt_v0, wt_v1]
    g_v = [g_v0, g_v1]
    out_v = [out_v0, out_v1]
    sem_in = [sem_in0, sem_in1]
    sem_g = [sem_g0, sem_g1]
    sem_out = [sem_out0, sem_out1]

    idx_hbm = [ix0, ix1, ix2, ix3]
    wt_hbm = [wx0, wx1, wx2, wx3]

    def stage(ch, b):
        ch = jnp.minimum(ch, NCHUNK - 1)
        for corner in range(4):
            pltpu.async_copy(idx_hbm[corner].at[pl.ds(ch * 2, 2)],
                             idx_v[b].at[pl.ds(corner * 2, 2)], sem_in[b])
            pltpu.async_copy(wt_hbm[corner].at[pl.ds(ch * 2, 2)],
                             wt_v[b].at[pl.ds(corner * 2, 2)], sem_in[b])

    def wait_stage(b):
        pltpu.make_async_copy(ix0.at[pl.ds(0, 8)], idx_v[b], sem_in[b]).wait()
        pltpu.make_async_copy(wx0.at[pl.ds(0, 8)], wt_v[b], sem_in[b]).wait()

    def gath(b):
        for j in range(8):
            pltpu.async_copy(
                table_hbm.at[idx_v[b].at[j]],
                g_v[b].at[pl.ds(j * 128, 128)], sem_g[b])

    def wait_gath(b):
        for j in range(8):
            pltpu.make_async_copy(table_hbm.at[pl.ds(0, 128)],
                                  g_v[b].at[pl.ds(j * 128, 128)], sem_g[b]).wait()

    def compute(b):
        wtb = wt_v[b]
        gvb = g_v[b]
        ovb = out_v[b]

        @pl.loop(0, 16)
        def _(i):
            hoff = (i & 7) * 16
            qq = i >> 3
            acc0 = jnp.zeros((16,), jnp.float32)
            acc1 = jnp.zeros((16,), jnp.float32)
            for corner in range(4):
                w16 = wtb[corner * 2 + qq, pl.ds(hoff, 16)]
                start = (corner * 2 + qq) * 128 + hoff
                for k in range(16):
                    wk = lax.gather(
                        w16, jnp.full((16, 1), k, jnp.int32),
                        lax.GatherDimensionNumbers(
                            offset_dims=(), collapsed_slice_dims=(0,),
                            start_index_map=(0,)),
                        (1,), mode=lax.GatherScatterMode.PROMISE_IN_BOUNDS)
                    g0 = gvb[start + k, pl.ds(0, 16)]
                    g1 = gvb[start + k, pl.ds(16, 16)]
                    acc0 = acc0 + wk * g0
                    acc1 = acc1 + wk * g1
            ovb[i, pl.ds(0, 16)] = acc0
            ovb[i, pl.ds(16, 16)] = acc1

    stage(base_ch, 0)
    stage(base_ch + 1, 1)
    pltpu.async_copy(out_v0, out_hbm.at[pl.ds(base_ch * 16, 16)], sem_out0)
    pltpu.async_copy(out_v1, out_hbm.at[pl.ds(base_ch * 16 + 16, 16)], sem_out1)
    wait_stage(0)
    gath(0)

    @pl.loop(0, CHPW, step=2)
    def _(t):
        for b in range(2):
            ch = base_ch + t + b
            nb = 1 - b
            wait_stage(nb)
            gath(nb)
            wait_gath(b)
            pltpu.make_async_copy(out_v[b], out_hbm.at[pl.ds(0, 16)],
                                  sem_out[b]).wait()
            compute(b)
            pltpu.async_copy(out_v[b], out_hbm.at[pl.ds(ch * 16, 16)], sem_out[b])
            stage(ch + 2, b)

    wait_stage(1)
    wait_gath(0)
    pltpu.make_async_copy(out_v0, out_hbm.at[pl.ds(0, 16)], sem_out0).wait()
    pltpu.make_async_copy(out_v1, out_hbm.at[pl.ds(0, 16)], sem_out1).wait()


@jax.jit
def kernel(query, query_pos, value, reference_points, spatial_shapes,
           W_so, b_so, W_aw, b_aw, W_v, b_v, W_o, b_o):
    del spatial_shapes
    sx, sy, saw, rx, ry, pool = _selection_matrices()

    q2 = query.reshape(NQ, EMBED)
    qp2 = query_pos.reshape(NQ, EMBED)
    refs = reference_points.reshape(NQ, NCAM * 2)

    grid_a = NQ // BQ
    full = lambda shape: pl.BlockSpec(shape, lambda i: (0, 0))
    *prep_outs, = pl.pallas_call(
        _prep_body,
        grid=(grid_a,),
        in_specs=[
            pl.BlockSpec((BQ, EMBED), lambda i: (i, 0)),
            pl.BlockSpec((BQ, EMBED), lambda i: (i, 0)),
            pl.BlockSpec((BQ, NCAM * 2), lambda i: (i, 0)),
            full((EMBED, 64)), full((1, 64)),
            full((EMBED, 32)), full((1, 32)),
            full((64, NLANE)), full((64, NLANE)), full((32, NLANE)),
            full((8, NLANE)), full((8, NLANE)), full((32, 32)),
        ],
        out_specs=[pl.BlockSpec((BQ, 128), lambda i: (i, 0))] * 8,
        out_shape=[jax.ShapeDtypeStruct((NQ, 128), jnp.int32)] * 4
                  + [jax.ShapeDtypeStruct((NQ, 128), jnp.float32)] * 4,
    )(q2, qp2, refs, W_so, b_so.reshape(1, 64), W_aw, b_aw.reshape(1, 32),
      jnp.asarray(sx), jnp.asarray(sy), jnp.asarray(saw),
      jnp.asarray(rx), jnp.asarray(ry), jnp.asarray(pool))

    v2 = value.reshape(NCAM * HW, EMBED)
    BV = 1024
    vproj = pl.pallas_call(
        _vproj_body,
        grid=(NCAM * HW // BV,),
        in_specs=[
            pl.BlockSpec((BV, EMBED), lambda i: (i, 0)),
            full((EMBED, EMBED)), full((1, EMBED)),
        ],
        out_specs=pl.BlockSpec((BV * 2, 128), lambda i: (i, 0)),
        out_shape=jax.ShapeDtypeStruct((NCAM * HW * 2, 128), jnp.float32),
    )(v2, W_v, b_v.reshape(1, EMBED))

    table = vproj.reshape(NCAM * HW * HEADS, DH)

    mesh = plsc.VectorSubcoreMesh(core_axis_name="c", subcore_axis_name="s")
    cp = pltpu.CompilerParams(needs_layout_passes=False,
                              use_tc_tiling_on_sc=False)
    msda = pl.kernel(
        _sc_kernel,
        out_type=jax.ShapeDtypeStruct((NPAIR, DH), jnp.float32),
        mesh=mesh,
        scratch_types=[
            pltpu.VMEM((8, 128), jnp.int32),
            pltpu.VMEM((8, 128), jnp.int32),
            pltpu.VMEM((8, 128), jnp.float32),
            pltpu.VMEM((8, 128), jnp.float32),
            pltpu.VMEM((QCH * NLANE, DH), jnp.float32),
            pltpu.VMEM((QCH * NLANE, DH), jnp.float32),
            pltpu.VMEM((16, DH), jnp.float32),
            pltpu.VMEM((16, DH), jnp.float32),
            pltpu.SemaphoreType.DMA,
            pltpu.SemaphoreType.DMA,
            pltpu.SemaphoreType.DMA,
            pltpu.SemaphoreType.DMA,
            pltpu.SemaphoreType.DMA,
            pltpu.SemaphoreType.DMA,
        ],
        compiler_params=cp,
    )(*prep_outs, table)

    m2 = msda.reshape(NQ, EMBED)
    out = pl.pallas_call(
        _out_body,
        grid=(NQ // BQ,),
        in_specs=[
            pl.BlockSpec((BQ, EMBED), lambda i: (i, 0)),
            full((EMBED, EMBED)), full((1, EMBED)),
            pl.BlockSpec((BQ, EMBED), lambda i: (i, 0)),
        ],
        out_specs=pl.BlockSpec((BQ, EMBED), lambda i: (i, 0)),
        out_shape=jax.ShapeDtypeStruct((NQ, EMBED), jnp.float32),
    )(m2, W_o, b_o.reshape(1, EMBED), q2)

    return out.reshape(1, NQ, EMBED)
